# baseline (speedup 1.0000x reference)
import functools

import jax
import jax.numpy as jnp
from jax import lax
from jax.experimental import pallas as pl
from jax.experimental.pallas import tpu as pltpu
from jax.experimental.pallas import tpu_sc as plsc

_NC = 2
_NS = 16
_NW = _NC * _NS
_NBUF = 4


def _sc_aggregate(x, src3d, dst3d, zrows, ib):
    np_, d = zrows.shape
    _, n_chunks, chunk = src3d.shape
    rpt = np_ // _NS

    mesh = plsc.VectorSubcoreMesh(
        core_axis_name="c", subcore_axis_name="s",
        num_cores=_NC, num_subcores=_NS)

    @functools.partial(
        pl.kernel, mesh=mesh,
        out_type=jax.ShapeDtypeStruct((_NC * np_, d), jnp.float32),
        compiler_params=pltpu.CompilerParams(use_tc_tiling_on_sc=False),
        scratch_types=[
            pltpu.VMEM((ib, chunk), jnp.int32),
            pltpu.VMEM((ib, chunk), jnp.int32),
        ] + [pltpu.VMEM((chunk, d), jnp.float32)] * _NBUF
          + [pltpu.VMEM_SHARED((np_, d), jnp.float32)]
          + [pltpu.SemaphoreType.DMA] * (2 * _NBUF))
    def k(x_hbm, src_hbm, dst_hbm, z_hbm, acc_out, src_v, dst_v, *rest):
        bufs = list(rest[:_NBUF])
        acc_s = rest[_NBUF]
        gsems = list(rest[_NBUF + 1:2 * _NBUF + 1])
        ssems = list(rest[2 * _NBUF + 1:])
        c = lax.axis_index("c")
        s = lax.axis_index("s")
        wid = s * _NC + c
        row0 = s * rpt
        pltpu.sync_copy(z_hbm.at[pl.ds(row0, rpt)], acc_s.at[pl.ds(row0, rpt)])
        plsc.subcore_barrier()

        def start_g(j, buf, sem):
            pltpu.async_copy(x_hbm.at[src_v.at[j]], buf, sem)

        def wait_g(buf, sem):
            pltpu.make_async_copy(x_hbm.at[src_v.at[0]], buf, sem).wait()

        def start_s(j, buf, sem):
            pltpu.async_copy(buf, acc_s.at[dst_v.at[j]], sem, add=True)

        def wait_s(buf, sem):
            pltpu.make_async_copy(buf, acc_s.at[dst_v.at[0]], sem).wait()

        def block_body(b, carry):
            pltpu.sync_copy(src_hbm.at[wid, pl.ds(b * ib, ib)], src_v)
            pltpu.sync_copy(dst_hbm.at[wid, pl.ds(b * ib, ib)], dst_v)
            for p in range(_NBUF):
                start_g(p, bufs[p], gsems[p])

            def loop_body(i, c2):
                j0 = _NBUF * i
                for p in range(_NBUF):
                    wait_g(bufs[p], gsems[p])
                    start_s(j0 + p, bufs[p], ssems[p])
                for p in range(_NBUF):
                    wait_s(bufs[p], ssems[p])
                    start_g(j0 + _NBUF + p, bufs[p], gsems[p])
                return c2

            lax.fori_loop(0, ib // _NBUF - 1, loop_body, 0)
            for p in range(_NBUF):
                wait_g(bufs[p], gsems[p])
                start_s(ib - _NBUF + p, bufs[p], ssems[p])
            for p in range(_NBUF):
                wait_s(bufs[p], ssems[p])
            return carry

        lax.fori_loop(0, n_chunks // ib, block_body, 0)

        plsc.subcore_barrier()
        pltpu.sync_copy(acc_s.at[pl.ds(row0, rpt)],
                        acc_out.at[pl.ds(c * np_ + row0, rpt)])

    return k(x, src3d, dst3d, zrows)


def _tc_combine(acc, degacc, xin, w_l, b_l, w_r, relu):
    d, d_out = w_l.shape
    npp = acc.shape[0] // 2
    bn = 632
    nb = npp // bn
    da = acc.shape[1]
    dx = xin.shape[1]

    dg = degacc.shape[1]

    def body(a0, a1, g0, g1, xr, wl, bl, wr, out):
        deg = g0[:, dg - 16:dg - 15] + g1[:, dg - 16:dg - 15]
        agg = (a0[:, :d] + a1[:, :d]) / jnp.maximum(deg, 1.0)
        r = jnp.dot(agg, wl[...], preferred_element_type=jnp.float32)
        r = r + bl[...] + jnp.dot(xr[:, :d], wr[...],
                                  preferred_element_type=jnp.float32)
        if relu:
            r = jnp.maximum(r, 0.0)
        out[...] = r

    return pl.pallas_call(
        body,
        grid=(nb,),
        in_specs=[
            pl.BlockSpec((bn, da), lambda i: (i, 0)),
            pl.BlockSpec((bn, da), lambda i: (i + nb, 0)),
            pl.BlockSpec((bn, dg), lambda i: (i, 0)),
            pl.BlockSpec((bn, dg), lambda i: (i + nb, 0)),
            pl.BlockSpec((bn, dx), lambda i: (i, 0)),
            pl.BlockSpec((d, d_out), lambda i: (0, 0)),
            pl.BlockSpec((1, d_out), lambda i: (0, 0)),
            pl.BlockSpec((d, d_out), lambda i: (0, 0)),
        ],
        out_specs=pl.BlockSpec((bn, d_out), lambda i: (i, 0)),
        out_shape=jax.ShapeDtypeStruct((npp, d_out), jnp.float32),
    )(acc, acc, degacc, degacc, xin, w_l, b_l.reshape(1, d_out), w_r)


def kernel(x, edge_index, W_l1, b_l1, W_r1, W_l2, b_l2, W_r2):
    n, d = x.shape
    e = edge_index.shape[1]
    e_per_w = e // _NW
    align = 8 * _NS
    np_ = ((n + align - 1) // align) * align

    def edges3d(chunk, ib):
        blk = chunk * ib
        e_pad = ((e_per_w + blk - 1) // blk) * blk
        n_chunks = e_pad // chunk
        srcw = jnp.pad(edge_index[0].reshape(_NW, e_per_w),
                       ((0, 0), (0, e_pad - e_per_w)))
        dstw = jnp.pad(edge_index[1].reshape(_NW, e_per_w),
                       ((0, 0), (0, e_pad - e_per_w)), constant_values=n)
        return (srcw.reshape(_NW, n_chunks, chunk),
                dstw.reshape(_NW, n_chunks, chunk))

    de = d + 16
    x_ext = jnp.pad(
        jnp.concatenate([x, jnp.ones((n, 16), jnp.float32)], axis=1),
        ((0, np_ - n), (0, 0)))
    src1, dst1 = edges3d(32, 24)
    acc1 = _sc_aggregate(x_ext, src1, dst1, jnp.zeros((np_, de), jnp.float32), 24)
    h = _tc_combine(acc1, acc1, x_ext, W_l1, b_l1, W_r1, relu=True)
    src2, dst2 = edges3d(40, 32)
    acc2 = _sc_aggregate(h, src2, dst2, jnp.zeros((np_, d), jnp.float32), 32)
    out = _tc_combine(acc2, acc1, h, W_l2, b_l2, W_r2, relu=False)
    return out[:n]


# speedup vs baseline: 1.6195x; 1.6195x over previous
import functools

import jax
import jax.numpy as jnp
from jax import lax
from jax.experimental import pallas as pl
from jax.experimental.pallas import tpu as pltpu
from jax.experimental.pallas import tpu_sc as plsc

_NC = 2
_NS = 16
_NW = _NC * _NS
_NBUF = 4


def _sc_aggregate(x, src3d, dst3d, zrows, ib):
    np_, d = zrows.shape
    _, n_chunks, chunk = src3d.shape
    rpt = np_ // _NS

    mesh = plsc.VectorSubcoreMesh(
        core_axis_name="c", subcore_axis_name="s",
        num_cores=_NC, num_subcores=_NS)

    @functools.partial(
        pl.kernel, mesh=mesh,
        out_type=jax.ShapeDtypeStruct((_NC * np_, d), jnp.float32),
        compiler_params=pltpu.CompilerParams(use_tc_tiling_on_sc=False),
        scratch_types=[
            pltpu.VMEM((ib, chunk), jnp.int32),
            pltpu.VMEM((ib, chunk), jnp.int32),
        ] + [pltpu.VMEM((chunk, d), jnp.float32)] * _NBUF
          + [pltpu.VMEM_SHARED((np_, d), jnp.float32)]
          + [pltpu.SemaphoreType.DMA] * (2 * _NBUF))
    def k(x_hbm, src_hbm, dst_hbm, z_hbm, acc_out, src_v, dst_v, *rest):
        bufs = list(rest[:_NBUF])
        acc_s = rest[_NBUF]
        gsems = list(rest[_NBUF + 1:2 * _NBUF + 1])
        ssems = list(rest[2 * _NBUF + 1:])
        c = lax.axis_index("c")
        s = lax.axis_index("s")
        wid = s * _NC + c
        row0 = s * rpt
        pltpu.sync_copy(z_hbm.at[pl.ds(row0, rpt)], acc_s.at[pl.ds(row0, rpt)])
        plsc.subcore_barrier()

        def start_g(j, buf, sem):
            pltpu.async_copy(x_hbm.at[src_v.at[j]], buf, sem)

        def wait_g(buf, sem):
            pltpu.make_async_copy(x_hbm.at[src_v.at[0]], buf, sem).wait()

        def start_s(j, buf, sem):
            pltpu.async_copy(buf, acc_s.at[dst_v.at[j]], sem, add=True)

        def wait_s(buf, sem):
            pltpu.make_async_copy(buf, acc_s.at[dst_v.at[0]], sem).wait()

        def block_body(b, carry):
            pltpu.sync_copy(src_hbm.at[wid, pl.ds(b * ib, ib)], src_v)
            pltpu.sync_copy(dst_hbm.at[wid, pl.ds(b * ib, ib)], dst_v)
            for p in range(_NBUF):
                start_g(p, bufs[p], gsems[p])

            def loop_body(i, c2):
                j0 = _NBUF * i
                for p in range(_NBUF):
                    wait_g(bufs[p], gsems[p])
                    start_s(j0 + p, bufs[p], ssems[p])
                for p in range(_NBUF):
                    wait_s(bufs[p], ssems[p])
                    start_g(j0 + _NBUF + p, bufs[p], gsems[p])
                return c2

            lax.fori_loop(0, ib // _NBUF - 1, loop_body, 0)
            for p in range(_NBUF):
                wait_g(bufs[p], gsems[p])
                start_s(ib - _NBUF + p, bufs[p], ssems[p])
            for p in range(_NBUF):
                wait_s(bufs[p], ssems[p])
            return carry

        lax.fori_loop(0, n_chunks // ib, block_body, 0)

        plsc.subcore_barrier()
        pltpu.sync_copy(acc_s.at[pl.ds(row0, rpt)],
                        acc_out.at[pl.ds(c * np_ + row0, rpt)])

    return k(x, src3d, dst3d, zrows)


def _tc_combine(acc, degacc, xin, w_l, b_l, w_r, relu):
    d, d_out = w_l.shape
    npp = acc.shape[0] // 2
    bn = 632
    nb = npp // bn
    da = acc.shape[1]
    dx = xin.shape[1]

    dg = degacc.shape[1]

    def body(a0, a1, g0, g1, xr, wl, bl, wr, out):
        deg = g0[:, dg - 16:dg - 15] + g1[:, dg - 16:dg - 15]
        agg = (a0[:, :d] + a1[:, :d]) / jnp.maximum(deg, 1.0)
        r = jnp.dot(agg, wl[...], preferred_element_type=jnp.float32)
        r = r + bl[...] + jnp.dot(xr[:, :d], wr[...],
                                  preferred_element_type=jnp.float32)
        if relu:
            r = jnp.maximum(r, 0.0)
        out[...] = r

    return pl.pallas_call(
        body,
        grid=(nb,),
        in_specs=[
            pl.BlockSpec((bn, da), lambda i: (i, 0)),
            pl.BlockSpec((bn, da), lambda i: (i + nb, 0)),
            pl.BlockSpec((bn, dg), lambda i: (i, 0)),
            pl.BlockSpec((bn, dg), lambda i: (i + nb, 0)),
            pl.BlockSpec((bn, dx), lambda i: (i, 0)),
            pl.BlockSpec((d, d_out), lambda i: (0, 0)),
            pl.BlockSpec((1, d_out), lambda i: (0, 0)),
            pl.BlockSpec((d, d_out), lambda i: (0, 0)),
        ],
        out_specs=pl.BlockSpec((bn, d_out), lambda i: (i, 0)),
        out_shape=jax.ShapeDtypeStruct((npp, d_out), jnp.float32),
    )(acc, acc, degacc, degacc, xin, w_l, b_l.reshape(1, d_out), w_r)


def kernel(x, edge_index, W_l1, b_l1, W_r1, W_l2, b_l2, W_r2):
    n, d = x.shape
    e = edge_index.shape[1]
    e_per_w = e // _NW
    align = 8 * _NS
    np_ = ((n + align - 1) // align) * align

    def edges3d(chunk, ib):
        blk = chunk * ib
        e_pad = ((e_per_w + blk - 1) // blk) * blk
        n_chunks = e_pad // chunk
        srcw = jnp.pad(edge_index[0].reshape(_NW, e_per_w),
                       ((0, 0), (0, e_pad - e_per_w)))
        dstw = jnp.pad(edge_index[1].reshape(_NW, e_per_w),
                       ((0, 0), (0, e_pad - e_per_w)), constant_values=n)
        return (srcw.reshape(_NW, n_chunks, chunk),
                dstw.reshape(_NW, n_chunks, chunk))

    de = d + 16
    x_ext = jnp.pad(
        jnp.concatenate([x, jnp.ones((n, 16), jnp.float32)], axis=1),
        ((0, np_ - n), (0, 0)))
    src1, dst1 = edges3d(32, 16)
    acc1 = _sc_aggregate(x_ext, src1, dst1, jnp.zeros((np_, de), jnp.float32),
                         16)
    h = _tc_combine(acc1, acc1, x_ext, W_l1, b_l1, W_r1, relu=True)
    src2, dst2 = edges3d(40, 16)
    acc2 = _sc_aggregate(h, src2, dst2, jnp.zeros((np_, d), jnp.float32), 32)
    out = _tc_combine(acc2, acc1, h, W_l2, b_l2, W_r2, relu=False)
    return out[:n]


# speedup vs baseline: 1.6203x; 1.0005x over previous
import functools

import jax
import jax.numpy as jnp
from jax import lax
from jax.experimental import pallas as pl
from jax.experimental.pallas import tpu as pltpu
from jax.experimental.pallas import tpu_sc as plsc

_NC = 2
_NS = 16
_NW = _NC * _NS
_NBUF = 4


def _sc_aggregate(x, src3d, dst3d, zrows, ib):
    np_, d = zrows.shape
    _, n_chunks, chunk = src3d.shape
    rpt = np_ // _NS

    mesh = plsc.VectorSubcoreMesh(
        core_axis_name="c", subcore_axis_name="s",
        num_cores=_NC, num_subcores=_NS)

    @functools.partial(
        pl.kernel, mesh=mesh,
        out_type=jax.ShapeDtypeStruct((_NC * np_, d), jnp.float32),
        compiler_params=pltpu.CompilerParams(use_tc_tiling_on_sc=False),
        scratch_types=[
            pltpu.VMEM((ib, chunk), jnp.int32),
            pltpu.VMEM((ib, chunk), jnp.int32),
        ] + [pltpu.VMEM((chunk, d), jnp.float32)] * _NBUF
          + [pltpu.VMEM_SHARED((np_, d), jnp.float32)]
          + [pltpu.SemaphoreType.DMA] * (2 * _NBUF))
    def k(x_hbm, src_hbm, dst_hbm, z_hbm, acc_out, src_v, dst_v, *rest):
        bufs = list(rest[:_NBUF])
        acc_s = rest[_NBUF]
        gsems = list(rest[_NBUF + 1:2 * _NBUF + 1])
        ssems = list(rest[2 * _NBUF + 1:])
        c = lax.axis_index("c")
        s = lax.axis_index("s")
        wid = s * _NC + c
        row0 = s * rpt
        pltpu.sync_copy(z_hbm.at[pl.ds(row0, rpt)], acc_s.at[pl.ds(row0, rpt)])
        plsc.subcore_barrier()

        def start_g(j, buf, sem):
            pltpu.async_copy(x_hbm.at[src_v.at[j]], buf, sem)

        def wait_g(buf, sem):
            pltpu.make_async_copy(x_hbm.at[src_v.at[0]], buf, sem).wait()

        def start_s(j, buf, sem):
            pltpu.async_copy(buf, acc_s.at[dst_v.at[j]], sem, add=True)

        def wait_s(buf, sem):
            pltpu.make_async_copy(buf, acc_s.at[dst_v.at[0]], sem).wait()

        def block_body(b, carry):
            pltpu.sync_copy(src_hbm.at[wid, pl.ds(b * ib, ib)], src_v)
            pltpu.sync_copy(dst_hbm.at[wid, pl.ds(b * ib, ib)], dst_v)
            for p in range(_NBUF):
                start_g(p, bufs[p], gsems[p])

            def loop_body(i, c2):
                j0 = _NBUF * i
                for p in range(_NBUF):
                    wait_g(bufs[p], gsems[p])
                    start_s(j0 + p, bufs[p], ssems[p])
                for p in range(_NBUF):
                    wait_s(bufs[p], ssems[p])
                    start_g(j0 + _NBUF + p, bufs[p], gsems[p])
                return c2

            lax.fori_loop(0, ib // _NBUF - 1, loop_body, 0)
            for p in range(_NBUF):
                wait_g(bufs[p], gsems[p])
                start_s(ib - _NBUF + p, bufs[p], ssems[p])
            for p in range(_NBUF):
                wait_s(bufs[p], ssems[p])
            return carry

        lax.fori_loop(0, n_chunks // ib, block_body, 0)

        plsc.subcore_barrier()
        pltpu.sync_copy(acc_s.at[pl.ds(row0, rpt)],
                        acc_out.at[pl.ds(c * np_ + row0, rpt)])

    return k(x, src3d, dst3d, zrows)


def _tc_combine(acc, degacc, xin, w_l, b_l, w_r, relu):
    d, d_out = w_l.shape
    npp = acc.shape[0] // 2
    bn = 632
    nb = npp // bn
    da = acc.shape[1]
    dx = xin.shape[1]

    dg = degacc.shape[1]

    def body(a0, a1, g0, g1, xr, wl, bl, wr, out):
        deg = g0[:, dg - 16:dg - 15] + g1[:, dg - 16:dg - 15]
        agg = (a0[:, :d] + a1[:, :d]) / jnp.maximum(deg, 1.0)
        r = jnp.dot(agg, wl[...], preferred_element_type=jnp.float32)
        r = r + bl[...] + jnp.dot(xr[:, :d], wr[...],
                                  preferred_element_type=jnp.float32)
        if relu:
            r = jnp.maximum(r, 0.0)
        out[...] = r

    return pl.pallas_call(
        body,
        grid=(nb,),
        in_specs=[
            pl.BlockSpec((bn, da), lambda i: (i, 0)),
            pl.BlockSpec((bn, da), lambda i: (i + nb, 0)),
            pl.BlockSpec((bn, dg), lambda i: (i, 0)),
            pl.BlockSpec((bn, dg), lambda i: (i + nb, 0)),
            pl.BlockSpec((bn, dx), lambda i: (i, 0)),
            pl.BlockSpec((d, d_out), lambda i: (0, 0)),
            pl.BlockSpec((1, d_out), lambda i: (0, 0)),
            pl.BlockSpec((d, d_out), lambda i: (0, 0)),
        ],
        out_specs=pl.BlockSpec((bn, d_out), lambda i: (i, 0)),
        out_shape=jax.ShapeDtypeStruct((npp, d_out), jnp.float32),
    )(acc, acc, degacc, degacc, xin, w_l, b_l.reshape(1, d_out), w_r)


def kernel(x, edge_index, W_l1, b_l1, W_r1, W_l2, b_l2, W_r2):
    n, d = x.shape
    e = edge_index.shape[1]
    e_per_w = e // _NW
    align = 8 * _NS
    np_ = ((n + align - 1) // align) * align

    def edges3d(chunk, ib):
        blk = chunk * ib
        e_pad = ((e_per_w + blk - 1) // blk) * blk
        n_chunks = e_pad // chunk
        srcw = jnp.pad(edge_index[0].reshape(_NW, e_per_w),
                       ((0, 0), (0, e_pad - e_per_w)))
        dstw = jnp.pad(edge_index[1].reshape(_NW, e_per_w),
                       ((0, 0), (0, e_pad - e_per_w)), constant_values=n)
        return (srcw.reshape(_NW, n_chunks, chunk),
                dstw.reshape(_NW, n_chunks, chunk))

    de = d + 16
    x_ext = jnp.pad(
        jnp.concatenate([x, jnp.ones((n, 16), jnp.float32)], axis=1),
        ((0, np_ - n), (0, 0)))
    src1, dst1 = edges3d(32, 16)
    acc1 = _sc_aggregate(x_ext, src1, dst1, jnp.zeros((np_, de), jnp.float32),
                         16)
    h = _tc_combine(acc1, acc1, x_ext, W_l1, b_l1, W_r1, relu=True)
    src2, dst2 = edges3d(40, 32)
    acc2 = _sc_aggregate(h, src2, dst2, jnp.zeros((np_, d), jnp.float32), 32)
    out = _tc_combine(acc2, acc1, h, W_l2, b_l2, W_r2, relu=False)
    return out[:n]
